# baseline (speedup 1.0000x reference)
import functools

import jax
import jax.numpy as jnp
from jax import lax
from jax.experimental import pallas as pl
from jax.experimental.pallas import tpu as pltpu
from jax.experimental.pallas import tpu_sc as plsc

B = 16384
K = 10
L = 16
NC, NS = 2, 16
NW = NC * NS
RPW = B // NW
GPW = RPW // L

_LN2 = 0.6931471805599453


def _ln(s):
    bits = lax.bitcast_convert_type(s, jnp.int32)
    e = lax.shift_right_logical(bits, 23) - 127
    m = lax.bitcast_convert_type(
        jnp.bitwise_or(jnp.bitwise_and(bits, 0x007FFFFF), 0x3F800000),
        jnp.float32)
    t = (m - 1.0) / (m + 1.0)
    t2 = t * t
    p = 1.0 / 9.0
    p = p * t2 + 1.0 / 7.0
    p = p * t2 + 1.0 / 5.0
    p = p * t2 + 1.0 / 3.0
    p = p * t2 + 1.0
    return 2.0 * t * p + e.astype(jnp.float32) * _LN2


def _body(x_hbm, c_hbm, y_hbm, lg_hbm, x_v, c_v, y_v, lg_v):
    wid = lax.axis_index("s") * NC + lax.axis_index("c")
    pltpu.sync_copy(x_hbm.at[pl.ds(wid * (RPW * 2), RPW * 2)], x_v)
    pltpu.sync_copy(c_hbm, c_v.at[pl.ds(0, 2 * K)])

    c_lo = c_v[pl.ds(0, L)]
    c_hi = c_v[pl.ds(L, L)]
    cs = [c_lo[i] if i < L else c_hi[i - L] for i in range(2 * K)]
    c0 = [cs[2 * j] for j in range(K)]
    c1 = [cs[2 * j + 1] for j in range(K)]
    iota = lax.iota(jnp.int32, L)
    gidx = iota * 2
    sidx = iota * K

    def group(i, carry):
        gb = i * (L * 2)
        vx0 = plsc.load_gather(x_v, [gidx + gb])
        vx1 = plsc.load_gather(x_v, [gidx + (gb + 1)])
        ls = []
        for j in range(K):
            d0 = vx0 - c0[j]
            d1 = vx1 - c1[j]
            ls.append(-(d0 * d0 + d1 * d1))
        m = ls[0]
        for j in range(1, K):
            m = jnp.maximum(m, ls[j])
        s = jnp.exp(ls[0] - m)
        for j in range(1, K):
            s = s + jnp.exp(ls[j] - m)
        tot = m + _ln(s)
        si = sidx + i * (L * K)
        for j in range(K):
            plsc.store_scatter(lg_v, [si + j], ls[j])
            plsc.store_scatter(y_v, [si + j], ls[j] - tot)
        return carry

    lax.fori_loop(0, GPW, group, 0)

    ob = wid * (RPW * K)
    pltpu.sync_copy(y_v, y_hbm.at[pl.ds(ob, RPW * K)])
    pltpu.sync_copy(lg_v, lg_hbm.at[pl.ds(ob, RPW * K)])


@functools.cache
def _sc_call():
    return pl.kernel(
        _body,
        out_type=(jax.ShapeDtypeStruct((B * K,), jnp.float32),
                  jax.ShapeDtypeStruct((B * K,), jnp.float32)),
        mesh=plsc.VectorSubcoreMesh(core_axis_name="c", subcore_axis_name="s",
                                    num_cores=NC, num_subcores=NS),
        scratch_types=[
            pltpu.VMEM((RPW * 2,), jnp.float32),
            pltpu.VMEM((2 * L,), jnp.float32),
            pltpu.VMEM((RPW * K,), jnp.float32),
            pltpu.VMEM((RPW * K,), jnp.float32),
        ],
        compiler_params=pltpu.CompilerParams(needs_layout_passes=False),
    )


@jax.jit
def kernel(x, embedding_weight):
    y_f, lg_f = _sc_call()(x.reshape(-1), embedding_weight.reshape(-1))
    return (y_f.reshape(B, K), lg_f.reshape(B, K))


# speedup vs baseline: 1.0569x; 1.0569x over previous
import functools

import jax
import jax.numpy as jnp
from jax import lax
from jax.experimental import pallas as pl
from jax.experimental.pallas import tpu as pltpu
from jax.experimental.pallas import tpu_sc as plsc

B = 16384
K = 10
L = 16
NC, NS = 2, 16
NW = NC * NS
RPW = B // NW
GPW = RPW // L

_LN2 = 0.6931471805599453


def _ln(s):
    bits = lax.bitcast_convert_type(s, jnp.int32)
    e = lax.shift_right_logical(bits, 23) - 127
    m = lax.bitcast_convert_type(
        jnp.bitwise_or(jnp.bitwise_and(bits, 0x007FFFFF), 0x3F800000),
        jnp.float32)
    t = (m - 1.0) / (m + 1.0)
    t2 = t * t
    p = 1.0 / 9.0
    p = p * t2 + 1.0 / 7.0
    p = p * t2 + 1.0 / 5.0
    p = p * t2 + 1.0 / 3.0
    p = p * t2 + 1.0
    return 2.0 * t * p + e.astype(jnp.float32) * _LN2


def _body(x_hbm, c_hbm, y_hbm, lg_hbm, x_v, c_v, y_v, lg_v):
    wid = lax.axis_index("s") * NC + lax.axis_index("c")
    pltpu.sync_copy(x_hbm.at[pl.ds(wid * RPW, RPW), :], x_v)
    pltpu.sync_copy(c_hbm, c_v)

    iota = lax.iota(jnp.int32, L)
    rows_a = lax.shift_right_logical(iota, 1)
    cols = jnp.bitwise_and(iota, 1)
    g_a = plsc.load_gather(c_v, [rows_a, cols])
    g_b = plsc.load_gather(c_v, [rows_a + 8, cols], mask=iota < (2 * K - L))
    cs = [g_a[i] for i in range(L)] + [g_b[i] for i in range(2 * K - L)]
    c0 = [cs[2 * j] for j in range(K)]
    c1 = [cs[2 * j + 1] for j in range(K)]

    col0 = jnp.bitwise_and(iota, 0)
    col1 = jnp.bitwise_or(col0, 1)
    jcol = [jnp.bitwise_or(col0, j) for j in range(K)]

    def group(i, carry):
        ridx = iota + i * L
        vx0 = plsc.load_gather(x_v, [ridx, col0])
        vx1 = plsc.load_gather(x_v, [ridx, col1])
        ls = []
        for j in range(K):
            d0 = vx0 - c0[j]
            d1 = vx1 - c1[j]
            ls.append(-(d0 * d0 + d1 * d1))
        m = ls[0]
        for j in range(1, K):
            m = jnp.maximum(m, ls[j])
        s = jnp.exp(ls[0] - m)
        for j in range(1, K):
            s = s + jnp.exp(ls[j] - m)
        tot = m + _ln(s)
        for j in range(K):
            plsc.store_scatter(lg_v, [ridx, jcol[j]], ls[j])
            plsc.store_scatter(y_v, [ridx, jcol[j]], ls[j] - tot)
        return carry

    lax.fori_loop(0, GPW, group, 0)

    pltpu.sync_copy(y_v, y_hbm.at[pl.ds(wid * RPW, RPW), :])
    pltpu.sync_copy(lg_v, lg_hbm.at[pl.ds(wid * RPW, RPW), :])


@functools.cache
def _sc_call():
    return pl.kernel(
        _body,
        out_type=(jax.ShapeDtypeStruct((B, K), jnp.float32),
                  jax.ShapeDtypeStruct((B, K), jnp.float32)),
        mesh=plsc.VectorSubcoreMesh(core_axis_name="c", subcore_axis_name="s",
                                    num_cores=NC, num_subcores=NS),
        scratch_types=[
            pltpu.VMEM((RPW, 2), jnp.float32),
            pltpu.VMEM((K, 2), jnp.float32),
            pltpu.VMEM((RPW, K), jnp.float32),
            pltpu.VMEM((RPW, K), jnp.float32),
        ],
        compiler_params=pltpu.CompilerParams(needs_layout_passes=False,
                                             use_tc_tiling_on_sc=False),
    )


@jax.jit
def kernel(x, embedding_weight):
    return _sc_call()(x, embedding_weight)


# speedup vs baseline: 2.8246x; 2.6726x over previous
import functools

import jax
import jax.numpy as jnp
from jax import lax
from jax.experimental import pallas as pl
from jax.experimental.pallas import tpu as pltpu
from jax.experimental.pallas import tpu_sc as plsc

B = 16384
K = 10
KP = 16
L = 16
NC, NS = 2, 16
NW = NC * NS
RPW = B // NW
GPW = RPW // L
CH = 128
HALF = (KP // 2) * B

_LN2 = 0.6931471805599453


def _ln(s):
    bits = lax.bitcast_convert_type(s, jnp.int32)
    e = lax.shift_right_logical(bits, 23) - 127
    m = lax.bitcast_convert_type(
        jnp.bitwise_or(jnp.bitwise_and(bits, 0x007FFFFF), 0x3F800000),
        jnp.float32)
    t = (m - 1.0) / (m + 1.0)
    t2 = t * t
    p = 1.0 / 9.0
    p = p * t2 + 1.0 / 7.0
    p = p * t2 + 1.0 / 5.0
    p = p * t2 + 1.0 / 3.0
    p = p * t2 + 1.0
    return 2.0 * t * p + e.astype(jnp.float32) * _LN2


def _body(x_hbm, c_hbm, y_hbm, lg_hbm, x_v, c_v, y_v, lg_v):
    wid = lax.axis_index("s") * NC + lax.axis_index("c")
    pltpu.sync_copy(x_hbm.at[pl.ds(wid * (RPW * 2), RPW * 2)], x_v)
    pltpu.sync_copy(c_hbm, c_v.at[pl.ds(0, 2 * K)])

    v_lo = c_v[pl.ds(0, L)]
    v_hi = c_v[pl.ds(L, L)]
    cf = [v_lo[i] for i in range(L)] + [v_hi[i] for i in range(2 * K - L)]
    c0 = cf[:K]
    c1 = cf[K:]

    def group(g, carry):
        chunk = lax.shift_right_logical(g, 3)
        sub = jnp.bitwise_and(g, 7)
        xoff = chunk * (2 * CH) + sub * L
        vx0 = x_v[pl.ds(xoff, L)]
        vx1 = x_v[pl.ds(xoff + CH, L)]
        ls = []
        for j in range(K):
            d0 = vx0 - c0[j]
            d1 = vx1 - c1[j]
            ls.append(-(d0 * d0 + d1 * d1))
        m = ls[0]
        for j in range(1, K):
            m = jnp.maximum(m, ls[j])
        s = jnp.exp(ls[0] - m)
        for j in range(1, K):
            s = s + jnp.exp(ls[j] - m)
        tot = m + _ln(s)
        lo = chunk * (8 * CH) + sub * L
        for j in range(K):
            off = lo + (j % 8) * CH + (0 if j < 8 else RPW * 8)
            lg_v[pl.ds(off, L)] = ls[j]
            y_v[pl.ds(off, L)] = ls[j] - tot
        return carry

    lax.fori_loop(0, GPW, group, 0)

    half = RPW * 8
    for hbm, v in ((y_hbm, y_v), (lg_hbm, lg_v)):
        pltpu.sync_copy(v.at[pl.ds(0, half)], hbm.at[pl.ds(wid * half, half)])
        pltpu.sync_copy(v.at[pl.ds(half, half)],
                        hbm.at[pl.ds(HALF + wid * half, half)])


@functools.cache
def _sc_call():
    return pl.kernel(
        _body,
        out_type=(jax.ShapeDtypeStruct((KP * B,), jnp.float32),
                  jax.ShapeDtypeStruct((KP * B,), jnp.float32)),
        mesh=plsc.VectorSubcoreMesh(core_axis_name="c", subcore_axis_name="s",
                                    num_cores=NC, num_subcores=NS),
        scratch_types=[
            pltpu.VMEM((RPW * 2,), jnp.float32),
            pltpu.VMEM((2 * L,), jnp.float32),
            pltpu.VMEM((RPW * KP,), jnp.float32),
            pltpu.VMEM((RPW * KP,), jnp.float32),
        ],
        compiler_params=pltpu.CompilerParams(needs_layout_passes=False,
                                             use_tc_tiling_on_sc=False),
    )


def _unphys(o):
    p = o.reshape(2, B // CH, 8, CH)
    return p.transpose(1, 3, 0, 2).reshape(B, KP)[:, :K]


@jax.jit
def kernel(x, embedding_weight):
    xb = x.reshape(B // CH, CH, 2).transpose(0, 2, 1).reshape(-1)
    wt = embedding_weight.T.reshape(-1)
    yb, lgb = _sc_call()(xb, wt)
    return _unphys(yb), _unphys(lgb)


# speedup vs baseline: 2.8478x; 1.0082x over previous
import functools

import jax
import jax.numpy as jnp
from jax import lax
from jax.experimental import pallas as pl
from jax.experimental.pallas import tpu as pltpu
from jax.experimental.pallas import tpu_sc as plsc

B = 16384
K = 10
KP = 16
L = 16
NC, NS = 2, 16
NW = NC * NS
RPW = B // NW
GPW = RPW // L
CH = 128
HALF = (KP // 2) * B

_LN2 = 0.6931471805599453


def _ln(s):
    bits = lax.bitcast_convert_type(s, jnp.int32)
    e = lax.shift_right_logical(bits, 23) - 127
    m = lax.bitcast_convert_type(
        jnp.bitwise_or(jnp.bitwise_and(bits, 0x007FFFFF), 0x3F800000),
        jnp.float32)
    t = (m - 1.0) / (m + 1.0)
    t2 = t * t
    p = 1.0 / 9.0
    p = p * t2 + 1.0 / 7.0
    p = p * t2 + 1.0 / 5.0
    p = p * t2 + 1.0 / 3.0
    p = p * t2 + 1.0
    return 2.0 * t * p + e.astype(jnp.float32) * _LN2


def _body(x_hbm, c_hbm, y_hbm, lg_hbm, x_v, c_v, y_v, lg_v):
    wid = lax.axis_index("s") * NC + lax.axis_index("c")
    pltpu.sync_copy(x_hbm.at[pl.ds(wid * (RPW * 2), RPW * 2)], x_v)
    pltpu.sync_copy(c_hbm.at[pl.ds(0, L)], c_v.at[pl.ds(0, L)])
    pltpu.sync_copy(c_hbm.at[pl.ds(CH, L)], c_v.at[pl.ds(L, L)])

    v0 = c_v[pl.ds(0, L)]
    v1 = c_v[pl.ds(L, L)]
    ca = [v0[j] + v0[j] for j in range(K)]
    cb = [v1[j] + v1[j] for j in range(K)]
    cn = [-(v0[j] * v0[j] + v1[j] * v1[j]) for j in range(K)]

    @plsc.parallel_loop(0, GPW, step=1, unroll=2)
    def group(g):
        chunk = lax.shift_right_logical(g, 3)
        sub = jnp.bitwise_and(g, 7)
        xoff = chunk * (2 * CH) + sub * L
        vx0 = x_v[pl.ds(xoff, L)]
        vx1 = x_v[pl.ds(xoff + CH, L)]
        r = vx0 * vx0 + vx1 * vx1
        ts = [vx0 * ca[j] + vx1 * cb[j] + cn[j] for j in range(K)]
        m = ts[0]
        for j in range(1, K):
            m = jnp.maximum(m, ts[j])
        s = jnp.exp(ts[0] - m)
        for j in range(1, K):
            s = s + jnp.exp(ts[j] - m)
        tot = m + _ln(s)
        lo = chunk * (8 * CH) + sub * L
        for j in range(K):
            off = lo + (j % 8) * CH + (0 if j < 8 else RPW * 8)
            lg_v[pl.ds(off, L)] = ts[j] - r
            y_v[pl.ds(off, L)] = ts[j] - tot

    half = RPW * 8
    for hbm, v in ((y_hbm, y_v), (lg_hbm, lg_v)):
        pltpu.sync_copy(v.at[pl.ds(0, half)], hbm.at[pl.ds(wid * half, half)])
        pltpu.sync_copy(v.at[pl.ds(half, half)],
                        hbm.at[pl.ds(HALF + wid * half, half)])


@functools.cache
def _sc_call():
    return pl.kernel(
        _body,
        out_type=(jax.ShapeDtypeStruct((KP * B,), jnp.float32),
                  jax.ShapeDtypeStruct((KP * B,), jnp.float32)),
        mesh=plsc.VectorSubcoreMesh(core_axis_name="c", subcore_axis_name="s",
                                    num_cores=NC, num_subcores=NS),
        scratch_types=[
            pltpu.VMEM((RPW * 2,), jnp.float32),
            pltpu.VMEM((2 * L,), jnp.float32),
            pltpu.VMEM((RPW * KP,), jnp.float32),
            pltpu.VMEM((RPW * KP,), jnp.float32),
        ],
        compiler_params=pltpu.CompilerParams(needs_layout_passes=False,
                                             use_tc_tiling_on_sc=False),
    )


def _unphys(o):
    p = o.reshape(2, B // CH, 8, CH)
    return p.transpose(1, 3, 0, 2).reshape(B, KP)[:, :K]


@jax.jit
def kernel(x, embedding_weight):
    xb = x.reshape(B // CH, CH, 2).transpose(0, 2, 1).reshape(-1)
    wp = jnp.pad(embedding_weight.T, ((0, 0), (0, CH - K))).reshape(-1)
    yb, lgb = _sc_call()(xb, wp)
    return _unphys(yb), _unphys(lgb)
